# baseline (speedup 1.0000x reference)
import jax
import jax.numpy as jnp
from jax.experimental import pallas as pl
from jax.experimental.pallas import tpu as pltpu

_B, _N, _C = 32, 8732, 21
_CH = 8
_NCHUNK = _B // _CH
_EPS = 1e-7


def _body(lam_ref, x_ref, y_ref, z_ref, sum_ref, cnt_ref):
    b = pl.program_id(0)

    @pl.when(b == 0)
    def _init():
        sum_ref[0, 0] = 0.0
        cnt_ref[0, 0] = 0.0

    lam = lam_ref[0]
    x = x_ref[0]
    y = y_ref[0]
    z = z_ref[0]

    mixed = lam * x + (1.0 - lam) * y + _EPS
    interp = z + _EPS
    p = (interp - mixed) * jnp.log(interp / mixed)

    lmax = jnp.max(x[1:], axis=0, keepdims=True)
    rmax = jnp.max(y[1:], axis=0, keepdims=True)
    mf = ((lmax > x[:1]) & (rmax > y[:1])).astype(jnp.float32)

    colsum = jnp.sum(p, axis=0, keepdims=True)
    sum_ref[0, 0] += jnp.sum(colsum * mf)
    cnt_ref[0, 0] += jnp.sum(mf)


def _chunk_partial(lam_arr, xt, yt, zt):
    return pl.pallas_call(
        _body,
        grid=(_CH,),
        in_specs=[
            pl.BlockSpec(memory_space=pltpu.SMEM),
            pl.BlockSpec((1, _C, _N), lambda b: (b, 0, 0)),
            pl.BlockSpec((1, _C, _N), lambda b: (b, 0, 0)),
            pl.BlockSpec((1, _C, _N), lambda b: (b, 0, 0)),
        ],
        out_specs=[
            pl.BlockSpec(memory_space=pltpu.SMEM),
            pl.BlockSpec(memory_space=pltpu.SMEM),
        ],
        out_shape=[
            jax.ShapeDtypeStruct((1, 1), jnp.float32),
            jax.ShapeDtypeStruct((1, 1), jnp.float32),
        ],
        compiler_params=pltpu.CompilerParams(
            dimension_semantics=("arbitrary",),
        ),
    )(lam_arr, xt, yt, zt)


def kernel(lam, conf, conf_flip, loc, loc_flip, conf_shuffle,
           conf_interpolation, loc_shuffle, loc_interpolation):
    lam_arr = jnp.asarray(lam, jnp.float32).reshape(1)
    half = _B // 2
    s = jnp.zeros((1, 1), jnp.float32)
    c = jnp.zeros((1, 1), jnp.float32)
    for k in range(_NCHUNK):
        lo = k * _CH
        swap_lo = (lo + half) % _B
        xt = jnp.transpose(conf[lo:lo + _CH], (0, 2, 1))
        yt = jnp.transpose(conf_shuffle[swap_lo:swap_lo + _CH], (0, 2, 1))
        zt = jnp.transpose(conf_interpolation[lo:lo + _CH], (0, 2, 1))
        ps, pc = _chunk_partial(lam_arr, xt, yt, zt)
        s = s + ps
        c = c + pc
    s0 = s[0, 0]
    c0 = c[0, 0]
    loss = jnp.where(c0 > 0.0, s0 / (2.0 * jnp.maximum(c0, 1.0)), 0.0)
    return loss


# speedup vs baseline: 1.2425x; 1.2425x over previous
import jax
import jax.numpy as jnp
from jax.experimental import pallas as pl
from jax.experimental.pallas import tpu as pltpu

_B, _N, _C = 32, 8732, 21
_CH = 16
_NCHUNK = _B // _CH
_EPS = 1e-7


def _body(lam_ref, x_ref, y_ref, z_ref, sum_ref, cnt_ref):
    b = pl.program_id(0)

    @pl.when(b == 0)
    def _init():
        sum_ref[0, 0] = 0.0
        cnt_ref[0, 0] = 0.0

    lam = lam_ref[0]
    x = x_ref[0]
    y = y_ref[0]
    z = z_ref[0]

    mixed = lam * x + (1.0 - lam) * y + _EPS
    interp = z + _EPS
    p = (interp - mixed) * jnp.log(interp / mixed)

    lmax = jnp.max(x[1:], axis=0, keepdims=True)
    rmax = jnp.max(y[1:], axis=0, keepdims=True)
    mf = ((lmax > x[:1]) & (rmax > y[:1])).astype(jnp.float32)

    colsum = jnp.sum(p, axis=0, keepdims=True)
    sum_ref[0, 0] += jnp.sum(colsum * mf)
    cnt_ref[0, 0] += jnp.sum(mf)


def _chunk_partial(lam_arr, xt, yt, zt):
    return pl.pallas_call(
        _body,
        grid=(_CH,),
        in_specs=[
            pl.BlockSpec(memory_space=pltpu.SMEM),
            pl.BlockSpec((1, _C, _N), lambda b: (b, 0, 0)),
            pl.BlockSpec((1, _C, _N), lambda b: (b, 0, 0)),
            pl.BlockSpec((1, _C, _N), lambda b: (b, 0, 0)),
        ],
        out_specs=[
            pl.BlockSpec(memory_space=pltpu.SMEM),
            pl.BlockSpec(memory_space=pltpu.SMEM),
        ],
        out_shape=[
            jax.ShapeDtypeStruct((1, 1), jnp.float32),
            jax.ShapeDtypeStruct((1, 1), jnp.float32),
        ],
        compiler_params=pltpu.CompilerParams(
            dimension_semantics=("arbitrary",),
        ),
    )(lam_arr, xt, yt, zt)


def kernel(lam, conf, conf_flip, loc, loc_flip, conf_shuffle,
           conf_interpolation, loc_shuffle, loc_interpolation):
    lam_arr = jnp.asarray(lam, jnp.float32).reshape(1)
    half = _B // 2
    s = jnp.zeros((1, 1), jnp.float32)
    c = jnp.zeros((1, 1), jnp.float32)
    for k in range(_NCHUNK):
        lo = k * _CH
        swap_lo = (lo + half) % _B
        xt = jnp.transpose(conf[lo:lo + _CH], (0, 2, 1))
        yt = jnp.transpose(conf_shuffle[swap_lo:swap_lo + _CH], (0, 2, 1))
        zt = jnp.transpose(conf_interpolation[lo:lo + _CH], (0, 2, 1))
        ps, pc = _chunk_partial(lam_arr, xt, yt, zt)
        s = s + ps
        c = c + pc
    s0 = s[0, 0]
    c0 = c[0, 0]
    loss = jnp.where(c0 > 0.0, s0 / (2.0 * jnp.maximum(c0, 1.0)), 0.0)
    return loss
